# Initial kernel scaffold; baseline (speedup 1.0000x reference)
import jax
import jax.numpy as jnp
from jax.experimental import pallas as pl


def kernel(indices, values):
    raise NotImplementedError("write your pallas kernel here")


# speedup vs baseline: 23.9250x; 23.9250x over previous
import functools

import jax
import jax.numpy as jnp
from jax import lax
from jax.experimental import pallas as pl
from jax.experimental.pallas import tpu as pltpu
from jax.experimental.pallas import tpu_sc as plsc

_N = 65536
_NNZ = 4194304
_NC = 2
_NS = 16
_NW = _NC * _NS
_SHARE = _NNZ // _NW
_W = 8192
_NWIN = _SHARE // _W
_L = 16
_SLABW = 16384
_NROUND = _N // _SLABW
_SLICE = _SLABW // _NS


def _sc_segment_sum(cols, vals):
    mesh = plsc.VectorSubcoreMesh(core_axis_name="c", subcore_axis_name="s")

    @functools.partial(
        pl.kernel,
        mesh=mesh,
        out_type=jax.ShapeDtypeStruct((_NC, _N), jnp.float32),
        compiler_params=pltpu.CompilerParams(needs_layout_passes=False),
        scratch_types=[
            pltpu.VMEM((_N,), jnp.float32),
            pltpu.VMEM((_W,), jnp.int32),
            pltpu.VMEM((_W,), jnp.float32),
            pltpu.VMEM((_SLICE,), jnp.float32),
            pltpu.VMEM((_SLICE,), jnp.float32),
            pltpu.VMEM_SHARED((_NS, _SLABW), jnp.float32),
        ],
    )
    def k(cols_hbm, vals_hbm, out_hbm, acc, idxw, valw, red, tmp, slab):
        c = lax.axis_index("c")
        s = lax.axis_index("s")
        wid = s * _NC + c
        base = wid * _SHARE

        zeros = jnp.zeros((_L,), jnp.float32)

        def zbody(i, carry):
            acc[pl.ds(i * _L, _L)] = zeros
            return carry

        lax.fori_loop(0, _N // _L, zbody, 0)

        def wbody(g, carry):
            off = base + g * _W
            pltpu.sync_copy(cols_hbm.at[pl.ds(off, _W)], idxw)
            pltpu.sync_copy(vals_hbm.at[pl.ds(off, _W)], valw)

            def ibody(i, icarry):
                iv = idxw[pl.ds(i * _L, _L)]
                vv = valw[pl.ds(i * _L, _L)]
                plsc.addupdate_scatter(acc, [iv], vv)
                return icarry

            lax.fori_loop(0, _W // _L, ibody, 0)
            return carry

        lax.fori_loop(0, _NWIN, wbody, 0)

        def round_body(r, carry):
            strip = r * _SLABW
            pltpu.sync_copy(acc.at[pl.ds(strip, _SLABW)], slab.at[s])
            plsc.subcore_barrier()

            off = s * _SLICE
            pltpu.sync_copy(slab.at[0, pl.ds(off, _SLICE)], red)

            def rbody(j, jcarry):
                pltpu.sync_copy(slab.at[j, pl.ds(off, _SLICE)], tmp)

                def abody(i, icarry):
                    sl = pl.ds(i * _L, _L)
                    red[sl] = red[sl] + tmp[sl]
                    return icarry

                lax.fori_loop(0, _SLICE // _L, abody, 0)
                return jcarry

            lax.fori_loop(1, _NS, rbody, 0)
            pltpu.sync_copy(red, out_hbm.at[c, pl.ds(strip + off, _SLICE)])
            plsc.subcore_barrier()
            return carry

        lax.fori_loop(0, _NROUND, round_body, 0)

    return k(cols, vals)


def _combine_partials(partials):
    p3 = partials.reshape(_NC, _N // 128, 128)

    def body(p_ref, o_ref):
        o_ref[...] = p_ref[0] + p_ref[1]

    out = pl.pallas_call(
        body,
        out_shape=jax.ShapeDtypeStruct((_N // 128, 128), jnp.float32),
    )(p3)
    return out.reshape(_N)


def kernel(indices, values):
    cols = indices[1].astype(jnp.int32)
    partials = _sc_segment_sum(cols, values)
    out = _combine_partials(partials)
    return (out, out)


# speedup vs baseline: 42.5088x; 1.7768x over previous
import functools

import jax
import jax.numpy as jnp
from jax import lax
from jax.experimental import pallas as pl
from jax.experimental.pallas import tpu as pltpu
from jax.experimental.pallas import tpu_sc as plsc

_N = 65536
_NNZ = 4194304
_NC = 2
_NS = 16
_NW = _NC * _NS
_SHARE = _NNZ // _NW
_W = 8192
_NWIN = _SHARE // _W
_L = 16
_SLABW = 16384
_NROUND = _N // _SLABW
_SLICE = _SLABW // _NS


def _sc_segment_sum(indices, vals):
    mesh = plsc.VectorSubcoreMesh(core_axis_name="c", subcore_axis_name="s")

    @functools.partial(
        pl.kernel,
        mesh=mesh,
        out_type=jax.ShapeDtypeStruct((_NC, _N), jnp.float32),
        compiler_params=pltpu.CompilerParams(needs_layout_passes=False),
        scratch_types=[
            pltpu.VMEM((_N,), jnp.float32),
            pltpu.VMEM((_W,), jnp.int32),
            pltpu.VMEM((_W,), jnp.int32),
            pltpu.VMEM((_W,), jnp.float32),
            pltpu.VMEM((_W,), jnp.float32),
            pltpu.VMEM((_SLICE,), jnp.float32),
            pltpu.VMEM((_SLICE,), jnp.float32),
            pltpu.VMEM_SHARED((_NS, _SLABW), jnp.float32),
            pltpu.SemaphoreType.DMA,
            pltpu.SemaphoreType.DMA,
            pltpu.SemaphoreType.DMA,
            pltpu.SemaphoreType.DMA,
        ],
    )
    def k(idx_hbm, vals_hbm, out_hbm, acc, idxw0, idxw1, valw0, valw1,
          red, tmp, slab, si0, si1, sv0, sv1):
        c = lax.axis_index("c")
        s = lax.axis_index("s")
        wid = s * _NC + c
        base = wid * _SHARE

        bufs = ((idxw0, valw0, si0, sv0), (idxw1, valw1, si1, sv1))

        def start_win(g, b):
            iw, vw, si, sv = bufs[b]
            off = base + g * _W
            ci = pltpu.make_async_copy(idx_hbm.at[1, pl.ds(off, _W)], iw, si)
            cv = pltpu.make_async_copy(vals_hbm.at[pl.ds(off, _W)], vw, sv)
            ci.start()
            cv.start()
            return ci, cv

        handles = [start_win(0, 0), None]

        zeros = jnp.zeros((_L,), jnp.float32)

        def zbody(i, carry):
            acc[pl.ds(i * _L, _L)] = zeros
            return carry

        lax.fori_loop(0, _N // _L, zbody, 0, unroll=16)

        for g in range(_NWIN):
            b = g % 2
            nb = (g + 1) % 2
            if g + 1 < _NWIN:
                handles[nb] = start_win(g + 1, nb)
            hi, hv = handles[b]
            hi.wait()
            hv.wait()
            iw, vw = bufs[b][0], bufs[b][1]

            def ibody(i, icarry, iw=iw, vw=vw):
                sl = pl.ds(i * _L, _L)
                plsc.addupdate_scatter(acc, [iw[sl]], vw[sl])
                return icarry

            lax.fori_loop(0, _W // _L, ibody, 0, unroll=8)

        for r in range(_NROUND):
            strip = r * _SLABW
            pltpu.sync_copy(acc.at[pl.ds(strip, _SLABW)], slab.at[s])
            plsc.subcore_barrier()

            off = s * _SLICE
            pltpu.sync_copy(slab.at[0, pl.ds(off, _SLICE)], red)

            def rbody(j, jcarry):
                pltpu.sync_copy(slab.at[j, pl.ds(off, _SLICE)], tmp)

                def abody(i, icarry):
                    sl = pl.ds(i * _L, _L)
                    red[sl] = red[sl] + tmp[sl]
                    return icarry

                lax.fori_loop(0, _SLICE // _L, abody, 0, unroll=8)
                return jcarry

            lax.fori_loop(1, _NS, rbody, 0)
            pltpu.sync_copy(red, out_hbm.at[c, pl.ds(strip + off, _SLICE)])
            plsc.subcore_barrier()

    return k(indices, vals)


def _combine_partials(partials):
    p3 = partials.reshape(_NC, _N // 128, 128)

    def body(p_ref, o_ref):
        o_ref[...] = p_ref[0] + p_ref[1]

    out = pl.pallas_call(
        body,
        out_shape=jax.ShapeDtypeStruct((_N // 128, 128), jnp.float32),
    )(p3)
    return out.reshape(_N)


def kernel(indices, values):
    if indices.dtype != jnp.int32:
        indices = indices.astype(jnp.int32)
    partials = _sc_segment_sum(indices, values)
    out = _combine_partials(partials)
    return (out, out)
